# baseline (speedup 1.0000x reference)
import functools

import jax
import jax.numpy as jnp
from jax import lax
from jax.experimental import pallas as pl
from jax.experimental.pallas import tpu as pltpu
from jax.experimental.pallas import tpu_sc as plsc

_LANES = 16
_IDX_CHUNK = 128


def _rsqrt_newton(x):
    i = lax.bitcast_convert_type(x, jnp.int32)
    y = lax.bitcast_convert_type(
        jnp.full_like(i, 0x5F3759DF) - lax.shift_right_logical(i, 1),
        jnp.float32)
    for _ in range(3):
        y = y * (jnp.float32(1.5) - jnp.float32(0.5) * x * y * y)
    return y


def _lane_shuffle(x, idx):
    return lax.gather(
        x, idx[:, None],
        dimension_numbers=lax.GatherDimensionNumbers(
            offset_dims=(), collapsed_slice_dims=(0,), start_index_map=(0,)),
        slice_sizes=(1,),
        mode=lax.GatherScatterMode.PROMISE_IN_BOUNDS)


@functools.lru_cache(maxsize=None)
def _build(B, D, n_chunks_total):
    info = plsc.get_sparse_core_info()
    nc, ns = info.num_cores, info.num_subcores
    nw = nc * ns
    b_per_w = B // nw
    ch_per_w = n_chunks_total // nw
    nvec = D // _LANES

    mesh = plsc.VectorSubcoreMesh(core_axis_name="c", subcore_axis_name="s")

    @functools.partial(
        pl.kernel,
        mesh=mesh,
        out_type=jax.ShapeDtypeStruct((B, D), jnp.float32),
        compiler_params=pltpu.CompilerParams(use_tc_tiling_on_sc=False),
        scratch_types=[
            pltpu.VMEM((ch_per_w, _IDX_CHUNK), jnp.int32),
            pltpu.VMEM((ch_per_w, _IDX_CHUNK), jnp.int32),
            pltpu.VMEM((b_per_w, D), jnp.float32),
            pltpu.VMEM((b_per_w, D), jnp.float32),
            pltpu.SemaphoreType.DMA,
        ],
    )
    def sc_kernel(heads_hbm, rels_hbm, etab_hbm, rtab_hbm, out_hbm,
                  hidx, ridx, erows, rrows, sem):
        wid = lax.axis_index("s") * nc + lax.axis_index("c")
        chunk_base = wid * ch_per_w
        row_base = wid * b_per_w

        pltpu.sync_copy(heads_hbm.at[pl.ds(chunk_base, ch_per_w)], hidx)
        pltpu.sync_copy(rels_hbm.at[pl.ds(chunk_base, ch_per_w)], ridx)

        copies = []
        for j in range(ch_per_w):
            dst = pl.ds(j * _IDX_CHUNK, _IDX_CHUNK)
            copies.append(pltpu.async_copy(
                etab_hbm.at[hidx.at[j]], erows.at[dst], sem))
            copies.append(pltpu.async_copy(
                rtab_hbm.at[ridx.at[j]], rrows.at[dst], sem))
        for c in copies:
            c.wait()

        lanes = lax.iota(jnp.int32, _LANES)
        perms = [lanes ^ p for p in (8, 4, 2, 1)]

        def row_fn(r, carry):
            vs = []
            ss = None
            for k in range(nvec):
                sl = pl.ds(k * _LANES, _LANES)
                v = erows[r, sl] + rrows[r, sl]
                vs.append(v)
                sq = v * v
                ss = sq if ss is None else ss + sq
            for p in perms:
                ss = ss + _lane_shuffle(ss, p)
            norm = ss * _rsqrt_newton(ss)
            inv = jnp.float32(1.0) / jnp.maximum(norm, jnp.float32(1e-12))
            for k, v in enumerate(vs):
                erows[r, pl.ds(k * _LANES, _LANES)] = v * inv
            return carry

        lax.fori_loop(0, b_per_w, row_fn, 0)

        pltpu.sync_copy(erows, out_hbm.at[pl.ds(row_base, b_per_w)])

    return sc_kernel


def kernel(heads, relations, entity_table, relation_table):
    B = heads.shape[0]
    D = entity_table.shape[1]
    n_chunks = B // _IDX_CHUNK
    heads2 = heads.reshape(n_chunks, _IDX_CHUNK).astype(jnp.int32)
    rels2 = relations.reshape(n_chunks, _IDX_CHUNK).astype(jnp.int32)
    fn = _build(B, D, n_chunks)
    return fn(heads2, rels2, entity_table, relation_table)


# speedup vs baseline: 2.5115x; 2.5115x over previous
import functools

import jax
import jax.numpy as jnp
from jax import lax
from jax.experimental import pallas as pl
from jax.experimental.pallas import tpu as pltpu
from jax.experimental.pallas import tpu_sc as plsc

_LANES = 16
_TILE = 8
_CH = 64


def _rsqrt_newton(x):
    i = lax.bitcast_convert_type(x, jnp.int32)
    y = lax.bitcast_convert_type(
        jnp.full_like(i, 0x5F3759DF) - lax.shift_right_logical(i, 1),
        jnp.float32)
    for _ in range(3):
        y = y * (jnp.float32(1.5) - jnp.float32(0.5) * x * y * y)
    return y


def _lane_shuffle(x, idx):
    return lax.gather(
        x, idx[:, None],
        dimension_numbers=lax.GatherDimensionNumbers(
            offset_dims=(), collapsed_slice_dims=(0,), start_index_map=(0,)),
        slice_sizes=(1,),
        mode=lax.GatherScatterMode.PROMISE_IN_BOUNDS)


@functools.lru_cache(maxsize=None)
def _build(B, D, n_etiles, n_rtiles):
    info = plsc.get_sparse_core_info()
    nc, ns = info.num_cores, info.num_subcores
    nw = nc * ns
    b_per_w = B // nw
    n_ch = b_per_w // _CH
    nvec = D // _LANES

    mesh = plsc.VectorSubcoreMesh(core_axis_name="c", subcore_axis_name="s")

    @functools.partial(
        pl.kernel,
        mesh=mesh,
        out_type=jax.ShapeDtypeStruct((B, D), jnp.float32),
        scratch_types=[
            pltpu.VMEM((b_per_w,), jnp.int32),
            pltpu.VMEM((b_per_w,), jnp.int32),
            pltpu.VMEM((_CH, D), jnp.float32),
            pltpu.VMEM((_CH, D), jnp.float32),
            pltpu.VMEM((_CH, D), jnp.float32),
            pltpu.SemaphoreType.DMA,
        ],
    )
    def sc_kernel(heads_hbm, rels_hbm, etab_hbm, rtab_hbm, out_hbm,
                  hidx, ridx, erow, rrow, outbuf, gsem):
        wid = lax.axis_index("s") * nc + lax.axis_index("c")
        row_base = wid * b_per_w

        pltpu.sync_copy(heads_hbm.at[pl.ds(row_base, b_per_w)], hidx)
        pltpu.sync_copy(rels_hbm.at[pl.ds(row_base, b_per_w)], ridx)

        lanes = lax.iota(jnp.int32, _LANES)
        perms = [lanes ^ p for p in (8, 4, 2, 1)]

        def chunk_fn(c, carry):
            cbase = c * _CH

            def fire(g, carry2):
                base = cbase + g * _LANES
                vh = hidx[pl.ds(base, _LANES)]
                vr = ridx[pl.ds(base, _LANES)]
                vht = lax.shift_right_logical(vh, 3)
                vhr = vh & (_TILE - 1)
                vrt = lax.shift_right_logical(vr, 3)
                vrr = vr & (_TILE - 1)
                for l in range(_LANES):
                    j = g * _LANES + l
                    pltpu.async_copy(
                        etab_hbm.at[vht[l], vhr[l]], erow.at[j], gsem)
                    pltpu.async_copy(
                        rtab_hbm.at[vrt[l], vrr[l]], rrow.at[j], gsem)
                return carry2

            lax.fori_loop(0, _CH // _LANES, fire, 0)
            pltpu.make_async_copy(
                out_hbm.at[pl.ds(0, _CH)], erow, gsem).wait()
            pltpu.make_async_copy(
                out_hbm.at[pl.ds(0, _CH)], rrow, gsem).wait()

            def row_fn(j, carry):
                vs = []
                ss = None
                for k in range(nvec):
                    sl = pl.ds(k * _LANES, _LANES)
                    v = erow[j, sl] + rrow[j, sl]
                    vs.append(v)
                    sq = v * v
                    ss = sq if ss is None else ss + sq
                for p in perms:
                    ss = ss + _lane_shuffle(ss, p)
                norm = ss * _rsqrt_newton(ss)
                inv = jnp.float32(1.0) / jnp.maximum(norm, jnp.float32(1e-12))
                for k, v in enumerate(vs):
                    outbuf[j, pl.ds(k * _LANES, _LANES)] = v * inv
                return carry

            lax.fori_loop(0, _CH, row_fn, 0)

            pltpu.sync_copy(outbuf, out_hbm.at[pl.ds(row_base + cbase, _CH)])
            return carry

        lax.fori_loop(0, n_ch, chunk_fn, 0)

    return sc_kernel


def kernel(heads, relations, entity_table, relation_table):
    B = heads.shape[0]
    N, D = entity_table.shape
    R = relation_table.shape[0]
    etab3 = entity_table.reshape(N // _TILE, _TILE, D)
    rtab3 = relation_table.reshape(R // _TILE, _TILE, D)
    fn = _build(B, D, N // _TILE, R // _TILE)
    return fn(heads.astype(jnp.int32), relations.astype(jnp.int32),
              etab3, rtab3)
